# baseline (speedup 1.0000x reference)
import functools

import jax
import jax.numpy as jnp
from jax import lax
from jax.experimental import pallas as pl
from jax.experimental.pallas import tpu as pltpu
from jax.experimental.pallas import tpu_sc as plsc

N = 100000
D = 32
R = 1000000
NW = 32
NPAD = 102400
PERW = 3200
TAIL0 = 999936
SEGCAP = 512
WIN = 512
NWIN = 64
WCAP = 96
SLOTS = NWIN * WCAP
OUTC = 100096

_mesh = plsc.VectorSubcoreMesh(core_axis_name="c", subcore_axis_name="s")

_I16 = lambda: lax.iota(jnp.int32, 16)


def _wid():
    return lax.axis_index("s") * 2 + lax.axis_index("c")


def _rank_and_ends(tmp_ref, keys):
    lanes = _I16()
    tmp_ref[pl.ds(0, 16)] = keys
    prev = plsc.load_gather(tmp_ref.at[pl.ds(0, 16)], [jnp.maximum(lanes - 1, 0)])
    nxt = plsc.load_gather(tmp_ref.at[pl.ds(0, 16)], [jnp.minimum(lanes + 1, 15)])
    is_start = (keys != prev) | (lanes == 0)
    start_pos = plsc.cummax(jnp.where(is_start, lanes, 0))
    rank = lanes - start_pos
    is_end = (keys != nxt) | (lanes == 15)
    return rank, is_end


@functools.partial(
    pl.kernel,
    mesh=_mesh,
    out_type=jax.ShapeDtypeStruct((NW, 256, 128), jnp.int32),
    compiler_params=pltpu.CompilerParams(needs_layout_passes=False),
    scratch_types=[
        pltpu.VMEM((PERW,), jnp.int32),
        pltpu.VMEM((NW, 8, 128), jnp.int32),
        pltpu.VMEM((128,), jnp.int32),
        pltpu.VMEM((32,), jnp.int32),
        pltpu.SemaphoreType.DMA,
    ],
)
def _phase_a(idx_hbm, segs_hbm, idx_v, stage_v, cnt_v, tmp_v, sem):
    w = _wid()
    lanes = _I16()
    pltpu.sync_copy(idx_hbm.at[pl.ds(pl.multiple_of(w * PERW, 128), PERW)], idx_v)

    sent = jnp.where(lanes & 1 == 1, -1, 1 << 26)

    def prefill_flat(j, c):
        base = j * 16
        plsc.store_scatter(
            stage_v,
            [jnp.full((16,), base // 1024, jnp.int32),
             (jnp.full((16,), base % 1024, jnp.int32) + lanes) // 128,
             (jnp.full((16,), base % 1024, jnp.int32) + lanes) % 128],
            sent,
        )
        return c

    lax.fori_loop(0, 2048, prefill_flat, 0)

    cnt_v[pl.ds(0, 16)] = jnp.zeros((16,), jnp.int32)
    cnt_v[pl.ds(16, 16)] = jnp.zeros((16,), jnp.int32)
    cnt_v[pl.ds(32, 16)] = jnp.zeros((16,), jnp.int32)
    cnt_v[pl.ds(48, 16)] = jnp.zeros((16,), jnp.int32)
    cnt_v[pl.ds(64, 16)] = jnp.zeros((16,), jnp.int32)
    cnt_v[pl.ds(80, 16)] = jnp.zeros((16,), jnp.int32)
    cnt_v[pl.ds(96, 16)] = jnp.zeros((16,), jnp.int32)
    cnt_v[pl.ds(112, 16)] = jnp.zeros((16,), jnp.int32)

    def group(g, c):
        idx = idx_v[pl.ds(g * 16, 16)]
        pos0 = w * PERW + g * 16
        valid = (pos0 + lanes) < N
        d = jnp.where(idx >= TAIL0, 31, idx >> 15)
        key = jnp.where(valid, d, 127)
        packed = (idx << 4) | lanes
        key_s, packed_s = plsc.sort_key_val(key, packed)
        idx_s = packed_s >> 4
        lane_s = packed_s & 15
        pos_s = pos0 + lane_s
        valid_s = key_s < 127
        rank, is_end = _rank_and_ends(tmp_v, key_s)
        base = plsc.load_gather(cnt_v, [key_s])
        slot = jnp.minimum(base + rank, SEGCAP - 1)
        off = slot * 2
        plsc.store_scatter(
            stage_v, [key_s, off >> 7, off & 127], idx_s, mask=valid_s
        )
        offp = off + 1
        plsc.store_scatter(
            stage_v, [key_s, offp >> 7, offp & 127], pos_s, mask=valid_s
        )
        plsc.store_scatter(cnt_v, [key_s], slot + 1, mask=is_end & valid_s)
        return c

    lax.fori_loop(0, PERW // 16, group, 0)

    copies = [
        pltpu.make_async_copy(
            stage_v.at[d], segs_hbm.at[d, pl.ds(pl.multiple_of(w * 8, 8), 8), :], sem
        )
        for d in range(NW)
    ]
    for cp in copies:
        cp.start()
    for cp in copies:
        cp.wait()


@functools.partial(
    pl.kernel,
    mesh=_mesh,
    out_type=(
        jax.ShapeDtypeStruct((NW, SLOTS // 4, 128), jnp.float32),
        jax.ShapeDtypeStruct((NW, SLOTS // 128, 128), jnp.int32),
    ),
    compiler_params=pltpu.CompilerParams(needs_layout_passes=False),
    scratch_types=[
        pltpu.VMEM((256, 128), jnp.int32),
        pltpu.VMEM((48, 128), jnp.int32),
        pltpu.VMEM((48, 128), jnp.int32),
        pltpu.VMEM((128,), jnp.int32),
        pltpu.VMEM((32,), jnp.int32),
        pltpu.VMEM((128, 128), jnp.float32),
        pltpu.VMEM((WCAP // 4, 128), jnp.float32),
        pltpu.SemaphoreType.DMA,
    ],
)
def _phase_b(tablet_hbm, tail_hbm, segs_hbm, rows_hbm, pos_hbm,
             seg_v, bidx_v, bpos_v, cnt_v, tmp_v, win_v, stage_v, sem):
    w = _wid()
    lanes = _I16()
    pltpu.sync_copy(segs_hbm.at[w], seg_v)

    sent = jnp.where(lanes & 1 == 1, -1, 1 << 26)

    def prefill(j, c):
        base = j * 16
        plsc.store_scatter(
            bidx_v,
            [jnp.full((16,), base // 128, jnp.int32),
             jnp.full((16,), base % 128, jnp.int32) + lanes],
            jnp.full((16,), 1 << 26, jnp.int32),
        )
        plsc.store_scatter(
            bpos_v,
            [jnp.full((16,), base // 128, jnp.int32),
             jnp.full((16,), base % 128, jnp.int32) + lanes],
            jnp.full((16,), -1, jnp.int32),
        )
        return c

    lax.fori_loop(0, SLOTS // 16, prefill, 0)

    for k in range(8):
        cnt_v[pl.ds(k * 16, 16)] = jnp.zeros((16,), jnp.int32)

    base_col = jnp.where(w == 31, TAIL0, w << 15)

    def bucket(g, c):
        woff = g * 32
        rows = jnp.full((16,), woff // 128, jnp.int32)
        cols = jnp.full((16,), woff % 128, jnp.int32) + lanes * 2
        idx = plsc.load_gather(seg_v, [rows, cols])
        pos = plsc.load_gather(seg_v, [rows, cols + 1])
        valid = pos >= 0
        key = jnp.where(valid, (idx - base_col) >> 9, 127)
        packed = (idx << 4) | lanes
        key_s, packed_s = plsc.sort_key_val(key, packed)
        idx_s = packed_s >> 4
        lane_s = packed_s & 15
        tmp_v[pl.ds(16, 16)] = pos
        pos_s = plsc.load_gather(tmp_v.at[pl.ds(16, 16)], [lane_s])
        valid_s = key_s < 127
        rank, is_end = _rank_and_ends(tmp_v, key_s)
        base = plsc.load_gather(cnt_v, [key_s])
        slot = jnp.minimum(base + rank, WCAP - 1)
        off = jnp.minimum(key_s, 63) * WCAP + slot
        plsc.store_scatter(bidx_v, [off >> 7, off & 127], idx_s, mask=valid_s)
        plsc.store_scatter(bpos_v, [off >> 7, off & 127], pos_s, mask=valid_s)
        plsc.store_scatter(cnt_v, [key_s], slot + 1, mask=is_end & valid_s)
        return c

    lax.fori_loop(0, SLOTS // 16, bucket, 0)

    nwin = jnp.where(w == 30, 33, jnp.where(w == 31, 1, NWIN))

    def sweep(i, c):
        @pl.when(w != 31)
        def _():
            for q in range(4):
                pltpu.sync_copy(
                    tablet_hbm.at[
                        :,
                        pl.ds(
                            pl.multiple_of(base_col + i * WIN + q * 128, 128),
                            128,
                        ),
                    ],
                    win_v.at[pl.ds(q * 32, 32), :],
                )

        @pl.when(w == 31)
        def _():
            pltpu.sync_copy(tail_hbm, win_v.at[pl.ds(0, 32), :])

        def group(g, c2):
            boff = i * WCAP + g * 16
            rows = jnp.full((16,), boff // 128, jnp.int32)
            cols = jnp.full((16,), boff % 128, jnp.int32) + lanes
            idx = plsc.load_gather(bidx_v, [rows, cols])
            pos = plsc.load_gather(bpos_v, [rows, cols])
            valid = pos >= 0
            col = jnp.clip(idx - base_col - i * WIN, 0, WIN - 1)
            sbase = (g * 16 + lanes) * D
            wrow = (col >> 7) * 32
            wcol = col & 127
            for f in range(D):
                v = plsc.load_gather(win_v, [wrow + f, wcol], mask=valid)
                soff = sbase + f
                plsc.store_scatter(
                    stage_v, [soff >> 7, soff & 127], v, mask=valid
                )
            return c2

        lax.fori_loop(0, WCAP // 16, group, 0)
        pltpu.sync_copy(
            stage_v, rows_hbm.at[w, pl.ds(pl.multiple_of(i * (WCAP // 4), 8), WCAP // 4), :]
        )
        return c

    lax.fori_loop(0, nwin, sweep, 0)
    pltpu.sync_copy(bpos_v, pos_hbm.at[w])


@functools.partial(
    pl.kernel,
    mesh=_mesh,
    out_type=jax.ShapeDtypeStruct((OUTC,), jnp.int32),
    compiler_params=pltpu.CompilerParams(use_tc_tiling_on_sc=False),
    scratch_types=[
        pltpu.VMEM((48, 128), jnp.int32),
        pltpu.VMEM((48, 128), jnp.int32),
        pltpu.SemaphoreType.DMA,
    ],
)
def _phase_cpre(pos_hbm, locmap_hbm, pos_v, slot_v, sem):
    w = _wid()
    lanes = _I16()
    pltpu.sync_copy(pos_hbm.at[w], pos_v)

    def fill(j, c):
        slot_v[j // 8, pl.ds((j % 8) * 16, 16)] = w * SLOTS + j * 16 + lanes
        return c

    lax.fori_loop(0, SLOTS // 16, fill, 0)

    def clean(j, c):
        row, q = j // 8, (j % 8) * 16
        p = pos_v[row, pl.ds(q, 16)]
        dump = N + ((j * 16 + lanes) & 63)
        pos_v[row, pl.ds(q, 16)] = jnp.where(p < 0, dump, p)
        return c

    lax.fori_loop(0, SLOTS // 16, clean, 0)

    copies = [
        pltpu.make_async_copy(
            slot_v.at[k], locmap_hbm.at[pos_v.at[k]], sem
        )
        for k in range(48)
    ]
    for cp in copies:
        cp.start()
    for cp in copies:
        cp.wait()


@functools.partial(
    pl.kernel,
    mesh=_mesh,
    out_type=jax.ShapeDtypeStruct((D, OUTC), jnp.float32),
    compiler_params=pltpu.CompilerParams(needs_layout_passes=False),
    scratch_types=[
        pltpu.VMEM((3200,), jnp.int32),
        pltpu.VMEM((3200,), jnp.int32),
        pltpu.VMEM((128, 128), jnp.float32),
        pltpu.VMEM((D, 128), jnp.float32),
        pltpu.SemaphoreType.DMA,
    ],
)
def _phase_cfin(locmap_hbm, rows_hbm, out_hbm, loc_v, qloc_v, gbuf_v,
                oblk_v, sem):
    w = _wid()
    lanes = _I16()
    nb = jnp.where(w < 14, 25, 24)
    sb = 24 * w + jnp.minimum(w, 14)
    p0 = sb * 128
    pltpu.sync_copy(locmap_hbm.at[pl.ds(pl.multiple_of(p0, 128), 3072)], loc_v.at[pl.ds(0, 3072)])

    @pl.when(w < 14)
    def _():
        pltpu.sync_copy(
            locmap_hbm.at[pl.ds(pl.multiple_of(p0 + 3072, 128), 128)], loc_v.at[pl.ds(3072, 128)]
        )

    def conv(j, c):
        v = loc_v[pl.ds(j * 16, 16)]
        qloc_v[pl.ds(j * 16, 16)] = jnp.minimum(v, NW * SLOTS - 1) >> 2
        return c

    lax.fori_loop(0, 200, conv, 0)

    def chunk(k, c):
        pltpu.async_copy(
            rows_hbm.at[qloc_v.at[pl.ds(k * 128, 128)]], gbuf_v, sem
        ).wait()

        def group(g, c2):
            loc = loc_v[pl.ds(k * 128 + g * 16, 16)]
            sub = (loc & 3) << 5
            rows = g * 16 + lanes
            for f in range(D):
                v = plsc.load_gather(gbuf_v, [rows, sub + f])
                plsc.store_scatter(
                    oblk_v, [jnp.full((16,), f, jnp.int32), rows], v
                )
            return c2

        lax.fori_loop(0, 8, group, 0)
        pltpu.sync_copy(oblk_v, out_hbm.at[:, pl.ds(pl.multiple_of(p0 + k * 128, 128), 128)])
        return c

    lax.fori_loop(0, nb, chunk, 0)


def kernel(node_feature, table):
    idx = node_feature[:, 0].astype(jnp.int32)
    pad = (jnp.arange(NPAD - N, dtype=jnp.int32) * 409) % TAIL0
    idxp = jnp.concatenate([idx, pad])
    tt = table.T
    tail = jnp.pad(tt[:, TAIL0:], ((0, 0), (0, 64)))
    segs = _phase_a(idxp)
    rows3, pos3 = _phase_b(tt, tail, segs)
    locmap = _phase_cpre(pos3)
    rows4 = rows3.reshape(NW * (SLOTS // 4), 128)
    outt = _phase_cfin(locmap, rows4)
    return outt.T[:N]


# speedup vs baseline: 41.2333x; 41.2333x over previous
import functools

import jax
import jax.numpy as jnp
from jax import lax
from jax.experimental import pallas as pl
from jax.experimental.pallas import tpu as pltpu
from jax.experimental.pallas import tpu_sc as plsc

N = 100000
D = 32
NC = 2
NS = 16
NW = NC * NS
CHUNK = 128
K = 25
B_PER_W = K * CHUNK
B_PAD = NW * B_PER_W

_mesh = plsc.VectorSubcoreMesh(core_axis_name="c", subcore_axis_name="s")


@functools.partial(
    pl.kernel,
    mesh=_mesh,
    out_type=jax.ShapeDtypeStruct((NW, K, CHUNK, D), jnp.float32),
    compiler_params=pltpu.CompilerParams(use_tc_tiling_on_sc=False),
    scratch_types=[
        pltpu.VMEM((K, CHUNK), jnp.int32),
        pltpu.VMEM((K, CHUNK, D), jnp.float32),
        pltpu.SemaphoreType.DMA,
    ],
)
def _gather_kernel(idx_hbm, table_hbm, out_hbm, idx_v, rows_v, sem):
    wid = lax.axis_index("s") * NC + lax.axis_index("c")
    pltpu.sync_copy(idx_hbm.at[pl.ds(wid * K, K), :], idx_v)

    def fire(j, carry):
        pltpu.make_async_copy(table_hbm.at[idx_v.at[j]], rows_v.at[j], sem).start()
        return carry

    lax.fori_loop(0, K, fire, 0)

    def drain(j, carry):
        pltpu.make_async_copy(table_hbm.at[idx_v.at[j]], rows_v.at[j], sem).wait()
        return carry

    lax.fori_loop(0, K, drain, 0)

    pltpu.sync_copy(rows_v, out_hbm.at[wid])


def kernel(node_feature, table):
    idx = node_feature[:, 0].astype(jnp.int32)
    idx = jnp.concatenate([idx, jnp.zeros((B_PAD - N,), jnp.int32)])
    out = _gather_kernel(idx.reshape(NW * K, CHUNK), table)
    return out.reshape(B_PAD, D)[:N]
